# baseline (speedup 1.0000x reference)
import functools

import jax
import jax.numpy as jnp
from jax.experimental import pallas as pl

DIM = 4096
NUM_EXPERTS = 64
TOP_K = 8
NUM_TOKENS = 32768
BLK_T = 512


def _router_block(x_ref, w_ref, rnd_ref, bias_ref, scores_out, idx_out,
                  cnt_out):
    i = pl.program_id(0)
    x = x_ref[...].astype(jnp.bfloat16)
    w = w_ref[...].astype(jnp.bfloat16)
    logits = jax.lax.dot_general(
        x, w, (((1,), (1,)), ((), ())),
        preferred_element_type=jnp.float32)
    s = (rnd_ref[...] - logits) + logits
    m = jnp.max(s, axis=1, keepdims=True)
    e = jnp.exp(s - m)
    p = e / jnp.sum(e, axis=1, keepdims=True)
    biased = p + bias_ref[...]

    lane = jax.lax.broadcasted_iota(jnp.int32, biased.shape, 1)
    neg_inf = jnp.float32(-jnp.inf)
    cnt = jnp.zeros((1, NUM_EXPERTS), jnp.int32)
    cur = biased
    for j in range(TOP_K):
        mx = jnp.max(cur, axis=1, keepdims=True)
        is_max = cur == mx
        idx = jnp.min(jnp.where(is_max, lane, NUM_EXPERTS), axis=1,
                      keepdims=True)
        onehot = lane == idx
        scores_out[:, j] = jnp.sum(jnp.where(onehot, p, 0.0), axis=1)
        idx_out[:, j] = idx[:, 0]
        cnt = cnt + jnp.sum(onehot.astype(jnp.int32), axis=0, keepdims=True)
        cur = jnp.where(onehot, neg_inf, cur)

    @pl.when(i == 0)
    def _init():
        cnt_out[...] = cnt

    @pl.when(i != 0)
    def _acc():
        cnt_out[...] = cnt_out[...] + cnt


@functools.partial(jax.jit, static_argnames=())
def kernel(x, expert_bias, W):
    n_tokens, dim = x.shape
    n_experts = W.shape[0]
    rnd = jax.random.normal(jax.random.key(42), (n_tokens, n_experts),
                            dtype=jnp.float32)
    bias2d = expert_bias.reshape(1, n_experts)
    grid = (n_tokens // BLK_T,)
    top_scores, idx, cnt = pl.pallas_call(
        _router_block,
        grid=grid,
        in_specs=[
            pl.BlockSpec((BLK_T, dim), lambda i: (i, 0)),
            pl.BlockSpec((n_experts, dim), lambda i: (0, 0)),
            pl.BlockSpec((BLK_T, n_experts), lambda i: (i, 0)),
            pl.BlockSpec((1, n_experts), lambda i: (0, 0)),
        ],
        out_specs=[
            pl.BlockSpec((BLK_T, TOP_K), lambda i: (i, 0)),
            pl.BlockSpec((BLK_T, TOP_K), lambda i: (i, 0)),
            pl.BlockSpec((1, n_experts), lambda i: (0, 0)),
        ],
        out_shape=[
            jax.ShapeDtypeStruct((n_tokens, TOP_K), jnp.float32),
            jax.ShapeDtypeStruct((n_tokens, TOP_K), jnp.int32),
            jax.ShapeDtypeStruct((1, n_experts), jnp.int32),
        ],
    )(x, W, rnd, bias2d)
    return top_scores, idx, cnt.reshape(n_experts)


# speedup vs baseline: 1.0471x; 1.0471x over previous
import functools

import jax
import jax.numpy as jnp
from jax.experimental import pallas as pl

DIM = 4096
NUM_EXPERTS = 64
TOP_K = 8
NUM_TOKENS = 32768
BLK_T = 1024


def _router_block(x_ref, w_ref, rnd_ref, bias_ref, scores_out, idx_out,
                  cnt_out):
    i = pl.program_id(0)
    x = x_ref[...].astype(jnp.bfloat16)
    w = w_ref[...].astype(jnp.bfloat16)
    logits = jax.lax.dot_general(
        x, w, (((1,), (1,)), ((), ())),
        preferred_element_type=jnp.float32)
    s = (rnd_ref[...] - logits) + logits
    m = jnp.max(s, axis=1, keepdims=True)
    e = jnp.exp(s - m)
    p = e / jnp.sum(e, axis=1, keepdims=True)
    biased = p + bias_ref[...]

    lane = jax.lax.broadcasted_iota(jnp.int32, biased.shape, 1)
    neg_inf = jnp.float32(-jnp.inf)
    cnt = jnp.zeros((1, NUM_EXPERTS), jnp.int32)
    cur = biased
    for j in range(TOP_K):
        mx = jnp.max(cur, axis=1, keepdims=True)
        is_max = cur == mx
        idx = jnp.min(jnp.where(is_max, lane, NUM_EXPERTS), axis=1,
                      keepdims=True)
        onehot = lane == idx
        scores_out[:, j] = jnp.sum(jnp.where(onehot, p, 0.0), axis=1)
        idx_out[:, j] = idx[:, 0]
        cnt = cnt + jnp.sum(onehot.astype(jnp.int32), axis=0, keepdims=True)
        cur = jnp.where(onehot, neg_inf, cur)

    @pl.when(i == 0)
    def _init():
        cnt_out[...] = cnt

    @pl.when(i != 0)
    def _acc():
        cnt_out[...] = cnt_out[...] + cnt


@functools.partial(jax.jit, static_argnames=())
def kernel(x, expert_bias, W):
    n_tokens, dim = x.shape
    n_experts = W.shape[0]
    rnd = jax.random.normal(jax.random.key(42), (n_tokens, n_experts),
                            dtype=jnp.float32)
    bias2d = expert_bias.reshape(1, n_experts)
    grid = (n_tokens // BLK_T,)
    top_scores, idx, cnt = pl.pallas_call(
        _router_block,
        grid=grid,
        in_specs=[
            pl.BlockSpec((BLK_T, dim), lambda i: (i, 0)),
            pl.BlockSpec((n_experts, dim), lambda i: (0, 0)),
            pl.BlockSpec((BLK_T, n_experts), lambda i: (i, 0)),
            pl.BlockSpec((1, n_experts), lambda i: (0, 0)),
        ],
        out_specs=[
            pl.BlockSpec((BLK_T, TOP_K), lambda i: (i, 0)),
            pl.BlockSpec((BLK_T, TOP_K), lambda i: (i, 0)),
            pl.BlockSpec((1, n_experts), lambda i: (0, 0)),
        ],
        out_shape=[
            jax.ShapeDtypeStruct((n_tokens, TOP_K), jnp.float32),
            jax.ShapeDtypeStruct((n_tokens, TOP_K), jnp.int32),
            jax.ShapeDtypeStruct((1, n_experts), jnp.int32),
        ],
    )(x, W, rnd, bias2d)
    return top_scores, idx, cnt.reshape(n_experts)
